# baseline (speedup 1.0000x reference)
import dataclasses

import jax
import jax.numpy as jnp
from jax import lax
from jax.experimental import pallas as pl
from jax.experimental.pallas import tpu as pltpu
from jax.experimental.pallas import tpu_sc as plsc

_B = 16384
_D = 32
_NC = 2
_NS = 16
_NW = _NC * _NS
_BPW = _B // _NW
_CH = 128
_NCH = _BPW // _CH
_L = 16


def _softplus(x):
    u = jnp.exp(-jnp.abs(x))
    z = u / (2.0 + u)
    z2 = z * z
    p = 2.0 * z * (1.0 + z2 * (1.0 / 3.0 + z2 * (0.2 + z2 * (1.0 / 7.0 + z2 * (1.0 / 9.0)))))
    return jnp.maximum(x, 0.0) + p


def _mirt_body(uid_hbm, qid_hbm, th_hbm, a_hbm, b16_hbm, out_hbm,
               uid_v, qid_v, bidx_v, th_v, a_v, b16_v, tp_v, out_v, sem):
    wid = lax.axis_index("s") * _NC + lax.axis_index("c")
    base = wid * _BPW

    idx_copies = []
    for j in range(_NCH):
        sl = pl.ds(base + j * _CH, _CH)
        idx_copies.append(pltpu.async_copy(uid_hbm.at[sl], uid_v.at[j], sem))
        idx_copies.append(pltpu.async_copy(qid_hbm.at[sl], qid_v.at[j], sem))
    for c in idx_copies:
        c.wait()

    for j in range(_NCH):
        for k in range(_CH // _L):
            qv = qid_v[j, pl.ds(k * _L, _L)]
            bidx_v[j, pl.ds(k * _L, _L)] = lax.shift_right_logical(qv, 4)

    gathers = []
    for j in range(_NCH):
        sl = pl.ds(j * _CH, _CH)
        gathers.append(pltpu.async_copy(th_hbm.at[uid_v.at[j]], th_v.at[sl], sem))
        gathers.append(pltpu.async_copy(a_hbm.at[qid_v.at[j]], a_v.at[sl], sem))
        gathers.append(pltpu.async_copy(b16_hbm.at[bidx_v.at[j]], b16_v.at[sl], sem))
    for g in gathers:
        g.wait()

    lane = lax.iota(jnp.int32, _L)

    @pl.loop(0, _BPW, step=_L)
    def _(r0):
        for j in range(_L):
            r = r0 + j
            p0 = _softplus(a_v[r, pl.ds(0, _L)]) * th_v[r, pl.ds(0, _L)]
            p1 = _softplus(a_v[r, pl.ds(_L, _L)]) * th_v[r, pl.ds(_L, _L)]
            plsc.store_scatter(tp_v, [lane, jnp.full((_L,), j, jnp.int32)], p0 + p1)
        acc = tp_v[0, pl.ds(0, _L)]
        for l in range(1, _L):
            acc = acc + tp_v[l, pl.ds(0, _L)]
        idx = r0 + lane
        qv = plsc.load_gather(qid_v, [lax.shift_right_logical(idx, 7),
                                      lax.bitwise_and(idx, _CH - 1)])
        bb = plsc.load_gather(b16_v, [idx, lax.bitwise_and(qv, _L - 1)])
        zz = acc - bb
        out_v[pl.ds(r0, _L)] = 1.0 / (1.0 + jnp.exp(-zz))

    pltpu.sync_copy(out_v, out_hbm.at[pl.ds(base, _BPW)])


def kernel(user_id, question_id, theta_table, a_table, b_table):
    mesh = plsc.VectorSubcoreMesh(core_axis_name="c", subcore_axis_name="s")
    cp = pltpu.CompilerParams()
    if "needs_layout_passes" in pltpu.CompilerParams.__dataclass_fields__:
        cp = dataclasses.replace(cp, needs_layout_passes=False)
    if "use_tc_tiling_on_sc" in pltpu.CompilerParams.__dataclass_fields__:
        cp = dataclasses.replace(cp, use_tc_tiling_on_sc=False)
    run = pl.kernel(
        _mirt_body,
        out_type=jax.ShapeDtypeStruct((_B,), jnp.float32),
        mesh=mesh,
        scratch_types=[
            pltpu.VMEM((_NCH, _CH), jnp.int32),
            pltpu.VMEM((_NCH, _CH), jnp.int32),
            pltpu.VMEM((_NCH, _CH), jnp.int32),
            pltpu.VMEM((_BPW, _D), jnp.float32),
            pltpu.VMEM((_BPW, _D), jnp.float32),
            pltpu.VMEM((_BPW, _L), jnp.float32),
            pltpu.VMEM((_L, _L), jnp.float32),
            pltpu.VMEM((_BPW,), jnp.float32),
            pltpu.SemaphoreType.DMA,
        ],
        compiler_params=cp,
    )
    return run(user_id.astype(jnp.int32), question_id.astype(jnp.int32),
               theta_table, a_table, b_table.reshape(b_table.shape[0] // _L, _L))
